# Initial kernel scaffold; baseline (speedup 1.0000x reference)
import jax
import jax.numpy as jnp
from jax.experimental import pallas as pl


def kernel(x, pos_emb, alpha, pe):
    raise NotImplementedError("write your pallas kernel here")


# speedup vs baseline: 1.8356x; 1.8356x over previous
import jax
import jax.numpy as jnp
from jax.experimental import pallas as pl
from jax.experimental.pallas import tpu as pltpu

_BS = 512


def _body(alpha_ref, x_ref, pos_ref, pe_ref, o_ref):
    a = jax.nn.sigmoid(alpha_ref[0, 0])
    combined = a * pos_ref[:] + (1.0 - a) * pe_ref[:]
    o_ref[:] = x_ref[:] + combined[None, :, :]


def kernel(x, pos_emb, alpha, pe):
    b, s, d = x.shape
    bs = _BS if s % _BS == 0 else s
    grid = (s // bs,)
    alpha2 = jnp.reshape(alpha, (1, 1))
    return pl.pallas_call(
        _body,
        grid=grid,
        in_specs=[
            pl.BlockSpec((1, 1), lambda i: (0, 0), memory_space=pltpu.SMEM),
            pl.BlockSpec((b, bs, d), lambda i: (0, i, 0)),
            pl.BlockSpec((bs, d), lambda i: (i, 0)),
            pl.BlockSpec((bs, d), lambda i: (i, 0)),
        ],
        out_specs=pl.BlockSpec((b, bs, d), lambda i: (0, i, 0)),
        out_shape=jax.ShapeDtypeStruct((b, s, d), x.dtype),
    )(alpha2, x, pos_emb[:s], pe[:s])
